# baseline (speedup 1.0000x reference)
import functools

import jax
import jax.numpy as jnp
from jax import lax
from jax.experimental import pallas as pl
from jax.experimental.pallas import tpu as pltpu
from jax.experimental.pallas import tpu_sc as plsc

N, D, OUT, E, K = 4096, 1024, 1024, 8, 2
BT = 512
NT = N // BT


def _e_of(s):
    return 3 * (s % 3) + s // 3


def _gating_body(x_ref, ea_ref, wg_ref, wep_ref, bep_ref,
                 gates_ref, idx_ref, loss_ref, acc_ref):
    t = pl.program_id(0)

    @pl.when(t == 0)
    def _():
        acc_ref[...] = jnp.zeros_like(acc_ref)

    logits = jnp.dot(x_ref[...], wg_ref[...],
                     preferred_element_type=jnp.float32)
    ea = ea_ref[...].astype(jnp.bfloat16).astype(jnp.float32)
    wep = wep_ref[...].astype(jnp.bfloat16).astype(jnp.float32)
    logits = logits + (ea[:, 0:1] * wep[0:1, :] + ea[:, 1:2] * wep[1:2, :])
    logits = logits + bep_ref[...]

    iota = lax.broadcasted_iota(jnp.int32, (BT, E), 1)
    m1 = jnp.max(logits, axis=1, keepdims=True)
    i1 = jnp.min(jnp.where(logits == m1, iota, E), axis=1, keepdims=True)
    masked = jnp.where(iota == i1, -jnp.inf, logits)
    m2 = jnp.max(masked, axis=1, keepdims=True)
    i2 = jnp.min(jnp.where(masked == m2, iota, E), axis=1, keepdims=True)
    tsm = jnp.exp(m2 - m1)
    denom = 1.0 + tsm
    g1 = 1.0 / denom
    g2 = tsm / denom

    gates_ref[...] = jnp.stack([jnp.broadcast_to(g1, (BT, 16)),
                                jnp.broadcast_to(g2, (BT, 16))], axis=0)
    rows = t * BT + lax.broadcasted_iota(jnp.int32, (BT, 1), 0)
    idx_ref[...] = jnp.concatenate([i1 * N + rows, i2 * N + rows], axis=1).T

    oh1 = (iota == i1).astype(jnp.float32)
    oh2 = (iota == i2).astype(jnp.float32)
    acc_ref[0:1, :] += jnp.sum(g1 * oh1 + g2 * oh2, axis=0, keepdims=True)
    acc_ref[1:2, :] += jnp.sum(oh1 + oh2, axis=0, keepdims=True)

    @pl.when(t == NT - 1)
    def _():
        def cv2(v):
            mean = jnp.mean(v)
            var = jnp.sum((v - mean) ** 2) / (E - 1)
            return var / (mean * mean + 1e-10)
        loss = 0.01 * (cv2(acc_ref[0:1, :]) + cv2(acc_ref[1:2, :]))
        loss_ref[...] = jnp.broadcast_to(loss, (1, 1))


def _gating(xb, edge_attr, w_gating, W_ep, b_ep):
    return pl.pallas_call(
        _gating_body,
        grid=(NT,),
        in_specs=[
            pl.BlockSpec((BT, D), lambda t: (t, 0)),
            pl.BlockSpec((BT, 2), lambda t: (t, 0)),
            pl.BlockSpec((D, E), lambda t: (0, 0)),
            pl.BlockSpec((2, E), lambda t: (0, 0)),
            pl.BlockSpec((1, E), lambda t: (0, 0)),
        ],
        out_specs=[
            pl.BlockSpec((K, BT, 16), lambda t: (0, t, 0)),
            pl.BlockSpec((K, BT), lambda t: (0, t)),
            pl.BlockSpec((1, 1), lambda t: (0, 0)),
        ],
        out_shape=[
            jax.ShapeDtypeStruct((K, N, 16), jnp.float32),
            jax.ShapeDtypeStruct((K, N), jnp.int32),
            jax.ShapeDtypeStruct((1, 1), jnp.float32),
        ],
        scratch_shapes=[pltpu.VMEM((2, E), jnp.float32)],
        compiler_params=pltpu.CompilerParams(
            dimension_semantics=("arbitrary",)),
    )(xb, edge_attr, w_gating.astype(jnp.bfloat16), W_ep, b_ep.reshape(1, E))


def _experts_body(xb_ref, w1_ref, b1_ref, w2_ref, b2_ref, w3_ref, b3_ref,
                  eo_ref):
    s = pl.program_id(0)
    t = pl.program_id(1)
    lyr = s // 3
    x = xb_ref[pl.ds(t * BT, BT), :]

    KC = 256

    def chained(h_prev, w_next_ref):
        acc = None
        for kc in range(D // KC):
            lo, hi = kc * KC, (kc + 1) * KC
            hb = jnp.maximum(h_prev[:, lo:hi], 0.0).astype(jnp.bfloat16)
            p = jnp.dot(hb, w_next_ref[0, lo:hi, :],
                        preferred_element_type=jnp.float32)
            acc = p if acc is None else acc + p
        return acc

    h1 = jnp.dot(x, w1_ref[0], preferred_element_type=jnp.float32)
    h1 = h1 + b1_ref[0]

    @pl.when(lyr == 0)
    def _():
        eo_ref[0] = h1

    @pl.when(lyr > 0)
    def _():
        h2 = chained(h1, w2_ref) + b2_ref[0]

        @pl.when(lyr == 1)
        def _():
            eo_ref[0] = h2

        @pl.when(lyr == 2)
        def _():
            eo_ref[0] = chained(h2, w3_ref) + b3_ref[0]


def _experts(xb, W1b, b1, W2b, b2, W3b, b3):
    wspec = pl.BlockSpec((1, D, OUT), lambda s, t: (_e_of(s), 0, 0))
    bspec = pl.BlockSpec((1, 1, OUT), lambda s, t: (_e_of(s), 0, 0))
    return pl.pallas_call(
        _experts_body,
        grid=(E, NT),
        in_specs=[
            pl.BlockSpec((N, D), lambda s, t: (0, 0)),
            wspec, bspec, wspec, bspec, wspec, bspec,
        ],
        out_specs=pl.BlockSpec((1, BT, OUT), lambda s, t: (_e_of(s), t, 0)),
        out_shape=jax.ShapeDtypeStruct((E, N, OUT), jnp.float32),
        compiler_params=pltpu.CompilerParams(
            dimension_semantics=("arbitrary", "arbitrary")),
    )(xb, W1b, b1.reshape(E, 1, OUT), W2b, b2.reshape(E, 1, OUT),
      W3b, b3.reshape(E, 1, OUT))


_NC, _NS = 2, 16
_NW = _NC * _NS
_BW = N // _NW
_CH = 16
_NCH = _BW // _CH


def _combine_sc_body(eo_hbm, gT_hbm, fT_hbm, out_hbm,
                     f1_v, f2_v, g1_v, g2_v, r1_v, r2_v, o_v, sem1, sem2):
    wid = lax.axis_index("s") * _NC + lax.axis_index("c")
    base = wid * _BW
    pltpu.sync_copy(fT_hbm.at[0, pl.ds(base, _BW)], f1_v)
    pltpu.sync_copy(fT_hbm.at[1, pl.ds(base, _BW)], f2_v)
    pltpu.sync_copy(gT_hbm.at[0, pl.ds(base * 16, _BW * 16)], g1_v)
    pltpu.sync_copy(gT_hbm.at[1, pl.ds(base * 16, _BW * 16)], g2_v)

    def chunk(c, carry):
        c0 = c * _CH
        cp1 = pltpu.async_copy(eo_hbm.at[f1_v[pl.ds(c0, _CH)]], r1_v, sem1)
        cp2 = pltpu.async_copy(eo_hbm.at[f2_v[pl.ds(c0, _CH)]], r2_v, sem2)
        cp1.wait()
        cp2.wait()

        def token(t, carry2):
            g1b = g1_v[pl.ds((c0 + t) * 16, 16)]
            g2b = g2_v[pl.ds((c0 + t) * 16, 16)]
            for h in range(OUT // 16):
                sl = pl.ds(h * 16, 16)
                o_v[t, sl] = g1b * r1_v[t, sl] + g2b * r2_v[t, sl]
            return carry2

        lax.fori_loop(0, _CH, token, 0)
        pltpu.sync_copy(o_v, out_hbm.at[pl.ds(base + c0, _CH)])
        return carry

    lax.fori_loop(0, _NCH, chunk, 0)


def _combine(eo, gRep, fT):
    gT = gRep.reshape(K, N * 16)
    eo2d = eo.reshape(E * N, OUT)
    mesh = plsc.VectorSubcoreMesh(core_axis_name="c", subcore_axis_name="s")
    import functools as _ft
    k = _ft.partial(
        pl.kernel, mesh=mesh,
        out_type=jax.ShapeDtypeStruct((N, OUT), jnp.float32),
        scratch_types=[
            pltpu.VMEM((_BW,), jnp.int32),
            pltpu.VMEM((_BW,), jnp.int32),
            pltpu.VMEM((_BW * 16,), jnp.float32),
            pltpu.VMEM((_BW * 16,), jnp.float32),
            pltpu.VMEM((_CH, OUT), jnp.float32),
            pltpu.VMEM((_CH, OUT), jnp.float32),
            pltpu.VMEM((_CH, OUT), jnp.float32),
            pltpu.SemaphoreType.DMA,
            pltpu.SemaphoreType.DMA,
        ])(_combine_sc_body)
    return k(eo2d, gT, fT)


def kernel(x, edge_attr, w_gating, W_ep, b_ep, W1, b1, W2, b2, W3, b3):
    xb = x.astype(jnp.bfloat16)
    gates, idx, loss = _gating(xb, edge_attr, w_gating, W_ep, b_ep)
    eo = _experts(xb, W1.astype(jnp.bfloat16), b1,
                  W2.astype(jnp.bfloat16), b2,
                  W3.astype(jnp.bfloat16), b3)
    final = _combine(eo, gates, idx)
    return final, eo, loss[0, 0]


# speedup vs baseline: 1.1151x; 1.1151x over previous
import functools

import jax
import jax.numpy as jnp
from jax import lax
from jax.experimental import pallas as pl
from jax.experimental.pallas import tpu as pltpu
from jax.experimental.pallas import tpu_sc as plsc

N, D, OUT, E, K = 4096, 1024, 1024, 8, 2
BT = 512
NT = N // BT


def _e_of(s):
    return 3 * (s % 3) + s // 3


def _gating_body(x_ref, ea_ref, wg_ref, wep_ref, bep_ref,
                 gates_ref, idx_ref, loss_ref, acc_ref):
    t = pl.program_id(0)

    @pl.when(t == 0)
    def _():
        acc_ref[...] = jnp.zeros_like(acc_ref)

    logits = jnp.dot(x_ref[...], wg_ref[...],
                     preferred_element_type=jnp.float32)
    ea = ea_ref[...].astype(jnp.bfloat16).astype(jnp.float32)
    wep = wep_ref[...].astype(jnp.bfloat16).astype(jnp.float32)
    logits = logits + (ea[:, 0:1] * wep[0:1, :] + ea[:, 1:2] * wep[1:2, :])
    logits = logits + bep_ref[...]

    iota = lax.broadcasted_iota(jnp.int32, (BT, E), 1)
    m1 = jnp.max(logits, axis=1, keepdims=True)
    i1 = jnp.min(jnp.where(logits == m1, iota, E), axis=1, keepdims=True)
    masked = jnp.where(iota == i1, -jnp.inf, logits)
    m2 = jnp.max(masked, axis=1, keepdims=True)
    i2 = jnp.min(jnp.where(masked == m2, iota, E), axis=1, keepdims=True)
    tsm = jnp.exp(m2 - m1)
    denom = 1.0 + tsm
    g1 = 1.0 / denom
    g2 = tsm / denom

    gates_ref[...] = jnp.stack([jnp.broadcast_to(g1, (BT, 16)),
                                jnp.broadcast_to(g2, (BT, 16))], axis=0)
    rows = t * BT + lax.broadcasted_iota(jnp.int32, (BT, 1), 0)
    idx_ref[...] = jnp.concatenate([i1 * N + rows, i2 * N + rows], axis=1).T

    oh1 = (iota == i1).astype(jnp.float32)
    oh2 = (iota == i2).astype(jnp.float32)
    acc_ref[0:1, :] += jnp.sum(g1 * oh1 + g2 * oh2, axis=0, keepdims=True)
    acc_ref[1:2, :] += jnp.sum(oh1 + oh2, axis=0, keepdims=True)

    @pl.when(t == NT - 1)
    def _():
        def cv2(v):
            mean = jnp.mean(v)
            var = jnp.sum((v - mean) ** 2) / (E - 1)
            return var / (mean * mean + 1e-10)
        loss = 0.01 * (cv2(acc_ref[0:1, :]) + cv2(acc_ref[1:2, :]))
        loss_ref[...] = jnp.broadcast_to(loss, (1, 1))


def _gating(xb, edge_attr, w_gating, W_ep, b_ep):
    return pl.pallas_call(
        _gating_body,
        grid=(NT,),
        in_specs=[
            pl.BlockSpec((BT, D), lambda t: (t, 0)),
            pl.BlockSpec((BT, 2), lambda t: (t, 0)),
            pl.BlockSpec((D, E), lambda t: (0, 0)),
            pl.BlockSpec((2, E), lambda t: (0, 0)),
            pl.BlockSpec((1, E), lambda t: (0, 0)),
        ],
        out_specs=[
            pl.BlockSpec((K, BT, 16), lambda t: (0, t, 0)),
            pl.BlockSpec((K, BT), lambda t: (0, t)),
            pl.BlockSpec((1, 1), lambda t: (0, 0)),
        ],
        out_shape=[
            jax.ShapeDtypeStruct((K, N, 16), jnp.float32),
            jax.ShapeDtypeStruct((K, N), jnp.int32),
            jax.ShapeDtypeStruct((1, 1), jnp.float32),
        ],
        scratch_shapes=[pltpu.VMEM((2, E), jnp.float32)],
        compiler_params=pltpu.CompilerParams(
            dimension_semantics=("arbitrary",)),
    )(xb, edge_attr, w_gating.astype(jnp.bfloat16), W_ep, b_ep.reshape(1, E))


def _experts_body(xb_ref, w1_ref, b1_ref, w2_ref, b2_ref, w3_ref, b3_ref,
                  eo_ref, w1b_s, w2b_s, w3b_s):
    s = pl.program_id(0)
    t = pl.program_id(1)
    lyr = s // 3

    @pl.when(t == 0)
    def _():
        w1b_s[...] = w1_ref[0].astype(jnp.bfloat16)

    @pl.when((t == 0) & (lyr > 0))
    def _():
        w2b_s[...] = w2_ref[0].astype(jnp.bfloat16)

    @pl.when((t == 0) & (lyr > 1))
    def _():
        w3b_s[...] = w3_ref[0].astype(jnp.bfloat16)

    x = xb_ref[pl.ds(t * BT, BT), :]

    KC = 256

    def chained(h_prev, wb_s):
        acc = None
        for kc in range(D // KC):
            lo, hi = kc * KC, (kc + 1) * KC
            hb = jnp.maximum(h_prev[:, lo:hi], 0.0).astype(jnp.bfloat16)
            p = jnp.dot(hb, wb_s[lo:hi, :],
                        preferred_element_type=jnp.float32)
            acc = p if acc is None else acc + p
        return acc

    h1 = jnp.dot(x, w1b_s[...], preferred_element_type=jnp.float32)
    h1 = h1 + b1_ref[0]

    @pl.when(lyr == 0)
    def _():
        eo_ref[0] = h1

    @pl.when(lyr > 0)
    def _():
        h2 = chained(h1, w2b_s) + b2_ref[0]

        @pl.when(lyr == 1)
        def _():
            eo_ref[0] = h2

        @pl.when(lyr == 2)
        def _():
            eo_ref[0] = chained(h2, w3b_s) + b3_ref[0]


def _experts(xb, W1, b1, W2, b2, W3, b3):
    bspec = pl.BlockSpec((1, 1, OUT), lambda s, t: (_e_of(s), 0, 0))
    w1spec = pl.BlockSpec((1, D, OUT), lambda s, t: (_e_of(s), 0, 0))
    w2spec = pl.BlockSpec((1, D, OUT),
                          lambda s, t: (_e_of(jnp.maximum(s, 3)), 0, 0))
    w3spec = pl.BlockSpec((1, D, OUT),
                          lambda s, t: (_e_of(jnp.maximum(s, 6)), 0, 0))
    return pl.pallas_call(
        _experts_body,
        grid=(E, NT),
        in_specs=[
            pl.BlockSpec((N, D), lambda s, t: (0, 0)),
            w1spec, bspec, w2spec, bspec, w3spec, bspec,
        ],
        out_specs=pl.BlockSpec((1, BT, OUT), lambda s, t: (_e_of(s), t, 0)),
        out_shape=jax.ShapeDtypeStruct((E, N, OUT), jnp.float32),
        scratch_shapes=[
            pltpu.VMEM((D, OUT), jnp.bfloat16),
            pltpu.VMEM((D, OUT), jnp.bfloat16),
            pltpu.VMEM((D, OUT), jnp.bfloat16),
        ],
        compiler_params=pltpu.CompilerParams(
            dimension_semantics=("arbitrary", "arbitrary")),
    )(xb, W1, b1.reshape(E, 1, OUT), W2, b2.reshape(E, 1, OUT),
      W3, b3.reshape(E, 1, OUT))


_NC, _NS = 2, 16
_NW = _NC * _NS
_BW = N // _NW
_CH = 16
_NCH = _BW // _CH


def _combine_sc_body(eo_hbm, gT_hbm, fT_hbm, out_hbm,
                     f1_v, f2_v, g1_v, g2_v, r1_v, r2_v, o_v, sem1, sem2):
    wid = lax.axis_index("s") * _NC + lax.axis_index("c")
    base = wid * _BW
    pltpu.sync_copy(fT_hbm.at[0, pl.ds(base, _BW)], f1_v)
    pltpu.sync_copy(fT_hbm.at[1, pl.ds(base, _BW)], f2_v)
    pltpu.sync_copy(gT_hbm.at[0, pl.ds(base * 16, _BW * 16)], g1_v)
    pltpu.sync_copy(gT_hbm.at[1, pl.ds(base * 16, _BW * 16)], g2_v)

    def chunk(c, carry):
        c0 = c * _CH
        cp1 = pltpu.async_copy(eo_hbm.at[f1_v[pl.ds(c0, _CH)]], r1_v, sem1)
        cp2 = pltpu.async_copy(eo_hbm.at[f2_v[pl.ds(c0, _CH)]], r2_v, sem2)
        cp1.wait()
        cp2.wait()

        def token(t, carry2):
            g1b = g1_v[pl.ds((c0 + t) * 16, 16)]
            g2b = g2_v[pl.ds((c0 + t) * 16, 16)]
            for h in range(OUT // 16):
                sl = pl.ds(h * 16, 16)
                o_v[t, sl] = g1b * r1_v[t, sl] + g2b * r2_v[t, sl]
            return carry2

        lax.fori_loop(0, _CH, token, 0)
        pltpu.sync_copy(o_v, out_hbm.at[pl.ds(base + c0, _CH)])
        return carry

    lax.fori_loop(0, _NCH, chunk, 0)


def _combine(eo, gRep, fT):
    gT = gRep.reshape(K, N * 16)
    eo2d = eo.reshape(E * N, OUT)
    mesh = plsc.VectorSubcoreMesh(core_axis_name="c", subcore_axis_name="s")
    import functools as _ft
    k = _ft.partial(
        pl.kernel, mesh=mesh,
        out_type=jax.ShapeDtypeStruct((N, OUT), jnp.float32),
        scratch_types=[
            pltpu.VMEM((_BW,), jnp.int32),
            pltpu.VMEM((_BW,), jnp.int32),
            pltpu.VMEM((_BW * 16,), jnp.float32),
            pltpu.VMEM((_BW * 16,), jnp.float32),
            pltpu.VMEM((_CH, OUT), jnp.float32),
            pltpu.VMEM((_CH, OUT), jnp.float32),
            pltpu.VMEM((_CH, OUT), jnp.float32),
            pltpu.SemaphoreType.DMA,
            pltpu.SemaphoreType.DMA,
        ])(_combine_sc_body)
    return k(eo2d, gT, fT)


def kernel(x, edge_attr, w_gating, W_ep, b_ep, W1, b1, W2, b2, W3, b3):
    xb = x.astype(jnp.bfloat16)
    gates, idx, loss = _gating(xb, edge_attr, w_gating, W_ep, b_ep)
    eo = _experts(xb, W1, b1, W2, b2, W3, b3)
    final = _combine(eo, gates, idx)
    return final, eo, loss[0, 0]


# speedup vs baseline: 1.3461x; 1.2072x over previous
import functools

import jax
import jax.numpy as jnp
from jax import lax
from jax.experimental import pallas as pl
from jax.experimental.pallas import tpu as pltpu
from jax.experimental.pallas import tpu_sc as plsc

N, D, OUT, E, K = 4096, 1024, 1024, 8, 2
BT = 1024
NT = N // BT


def _e_of(s):
    return 3 * (s % 3) + s // 3


def _gating_body(x_ref, ea_ref, wg_ref, wep_ref, bep_ref,
                 xb_ref, gates_ref, idx_ref, loss_ref, acc_ref):
    t = pl.program_id(0)

    @pl.when(t == 0)
    def _():
        acc_ref[...] = jnp.zeros_like(acc_ref)

    xb = x_ref[...].astype(jnp.bfloat16)
    xb_ref[...] = xb
    logits = jnp.dot(xb, wg_ref[...],
                     preferred_element_type=jnp.float32)
    ea = ea_ref[...].astype(jnp.bfloat16).astype(jnp.float32)
    wep = wep_ref[...].astype(jnp.bfloat16).astype(jnp.float32)
    logits = logits + (ea[:, 0:1] * wep[0:1, :] + ea[:, 1:2] * wep[1:2, :])
    logits = logits + bep_ref[...]

    iota = lax.broadcasted_iota(jnp.int32, (BT, E), 1)
    m1 = jnp.max(logits, axis=1, keepdims=True)
    i1 = jnp.min(jnp.where(logits == m1, iota, E), axis=1, keepdims=True)
    masked = jnp.where(iota == i1, -jnp.inf, logits)
    m2 = jnp.max(masked, axis=1, keepdims=True)
    i2 = jnp.min(jnp.where(masked == m2, iota, E), axis=1, keepdims=True)
    tsm = jnp.exp(m2 - m1)
    denom = 1.0 + tsm
    g1 = 1.0 / denom
    g2 = tsm / denom

    gates_ref[...] = jnp.stack([jnp.broadcast_to(g1, (BT, 16)),
                                jnp.broadcast_to(g2, (BT, 16))], axis=0)
    rows = t * BT + lax.broadcasted_iota(jnp.int32, (BT, 1), 0)
    idx_ref[...] = jnp.concatenate([i1 * N + rows, i2 * N + rows], axis=1).T

    oh1 = (iota == i1).astype(jnp.float32)
    oh2 = (iota == i2).astype(jnp.float32)
    acc_ref[0:1, :] += jnp.sum(g1 * oh1 + g2 * oh2, axis=0, keepdims=True)
    acc_ref[1:2, :] += jnp.sum(oh1 + oh2, axis=0, keepdims=True)

    @pl.when(t == NT - 1)
    def _():
        def cv2(v):
            mean = jnp.mean(v)
            var = jnp.sum((v - mean) ** 2) / (E - 1)
            return var / (mean * mean + 1e-10)
        loss = 0.01 * (cv2(acc_ref[0:1, :]) + cv2(acc_ref[1:2, :]))
        loss_ref[...] = jnp.broadcast_to(loss, (1, 1))


def _gating(x, edge_attr, w_gating, W_ep, b_ep):
    return pl.pallas_call(
        _gating_body,
        grid=(NT,),
        in_specs=[
            pl.BlockSpec((BT, D), lambda t: (t, 0)),
            pl.BlockSpec((BT, 2), lambda t: (t, 0)),
            pl.BlockSpec((D, E), lambda t: (0, 0)),
            pl.BlockSpec((2, E), lambda t: (0, 0)),
            pl.BlockSpec((1, E), lambda t: (0, 0)),
        ],
        out_specs=[
            pl.BlockSpec((BT, D), lambda t: (t, 0)),
            pl.BlockSpec((K, BT, 16), lambda t: (0, t, 0)),
            pl.BlockSpec((K, BT), lambda t: (0, t)),
            pl.BlockSpec((1, 1), lambda t: (0, 0)),
        ],
        out_shape=[
            jax.ShapeDtypeStruct((N, D), jnp.bfloat16),
            jax.ShapeDtypeStruct((K, N, 16), jnp.float32),
            jax.ShapeDtypeStruct((K, N), jnp.int32),
            jax.ShapeDtypeStruct((1, 1), jnp.float32),
        ],
        scratch_shapes=[pltpu.VMEM((2, E), jnp.float32)],
        compiler_params=pltpu.CompilerParams(
            dimension_semantics=("arbitrary",)),
    )(x, edge_attr, w_gating.astype(jnp.bfloat16), W_ep, b_ep.reshape(1, E))


def _experts_body(xb_ref, w1_ref, b1_ref, w2_ref, b2_ref, w3_ref, b3_ref,
                  eo_ref, w1b_s, w2b_s, w3b_s):
    s = pl.program_id(0)
    t = pl.program_id(1)
    lyr = s // 3

    @pl.when(t == 0)
    def _():
        w1b_s[...] = w1_ref[0].astype(jnp.bfloat16)

    @pl.when((t == 0) & (lyr > 0))
    def _():
        w2b_s[...] = w2_ref[0].astype(jnp.bfloat16)

    @pl.when((t == 0) & (lyr > 1))
    def _():
        w3b_s[...] = w3_ref[0].astype(jnp.bfloat16)

    x = xb_ref[pl.ds(t * BT, BT), :]

    def chained(h_prev, wb_s):
        hb = jnp.maximum(h_prev, 0.0).astype(jnp.bfloat16)
        return jnp.dot(hb, wb_s[...], preferred_element_type=jnp.float32)

    h1 = jnp.dot(x, w1b_s[...], preferred_element_type=jnp.float32)
    h1 = h1 + b1_ref[0]

    @pl.when(lyr == 0)
    def _():
        eo_ref[0] = h1

    @pl.when(lyr > 0)
    def _():
        h2 = chained(h1, w2b_s) + b2_ref[0]

        @pl.when(lyr == 1)
        def _():
            eo_ref[0] = h2

        @pl.when(lyr == 2)
        def _():
            eo_ref[0] = chained(h2, w3b_s) + b3_ref[0]


def _experts(xb, W1, b1, W2, b2, W3, b3):
    bspec = pl.BlockSpec((1, 1, OUT), lambda s, t: (_e_of(s), 0, 0))
    w1spec = pl.BlockSpec((1, D, OUT), lambda s, t: (_e_of(s), 0, 0))
    w2spec = pl.BlockSpec((1, D, OUT),
                          lambda s, t: (_e_of(jnp.maximum(s, 3)), 0, 0))
    w3spec = pl.BlockSpec((1, D, OUT),
                          lambda s, t: (_e_of(jnp.maximum(s, 6)), 0, 0))
    return pl.pallas_call(
        _experts_body,
        grid=(E, NT),
        in_specs=[
            pl.BlockSpec((N, D), lambda s, t: (0, 0)),
            w1spec, bspec, w2spec, bspec, w3spec, bspec,
        ],
        out_specs=pl.BlockSpec((1, BT, OUT), lambda s, t: (_e_of(s), t, 0)),
        out_shape=jax.ShapeDtypeStruct((E, N, OUT), jnp.float32),
        scratch_shapes=[
            pltpu.VMEM((D, OUT), jnp.bfloat16),
            pltpu.VMEM((D, OUT), jnp.bfloat16),
            pltpu.VMEM((D, OUT), jnp.bfloat16),
        ],
        compiler_params=pltpu.CompilerParams(
            dimension_semantics=("arbitrary", "arbitrary")),
    )(xb, W1, b1.reshape(E, 1, OUT), W2, b2.reshape(E, 1, OUT),
      W3, b3.reshape(E, 1, OUT))


_NC, _NS = 2, 16
_NW = _NC * _NS
_BW = N // _NW
_CH = 16
_NCH = _BW // _CH


def _combine_sc_body(eo_hbm, gT_hbm, fT_hbm, out_hbm,
                     f1_v, f2_v, g1_v, g2_v,
                     r1a_v, r2a_v, r1b_v, r2b_v, oa_v, ob_v,
                     s1a, s2a, s1b, s2b, soa, sob):
    wid = lax.axis_index("s") * _NC + lax.axis_index("c")
    base = wid * _BW
    pltpu.sync_copy(fT_hbm.at[0, pl.ds(base, _BW)], f1_v)
    pltpu.sync_copy(fT_hbm.at[1, pl.ds(base, _BW)], f2_v)
    pltpu.sync_copy(gT_hbm.at[0, pl.ds(base * 16, _BW * 16)], g1_v)
    pltpu.sync_copy(gT_hbm.at[1, pl.ds(base * 16, _BW * 16)], g2_v)

    bufs = ((r1a_v, r2a_v, s1a, s2a), (r1b_v, r2b_v, s1b, s2b))
    obufs = ((oa_v, soa), (ob_v, sob))

    def start(c):
        r1_v, r2_v, sem1, sem2 = bufs[c % 2]
        c0 = c * _CH
        cp1 = pltpu.async_copy(eo_hbm.at[f1_v[pl.ds(c0, _CH)]], r1_v, sem1)
        cp2 = pltpu.async_copy(eo_hbm.at[f2_v[pl.ds(c0, _CH)]], r2_v, sem2)
        return cp1, cp2

    cps = {0: start(0)}
    st_cps = {}
    for c in range(_NCH):
        r1_v, r2_v, _, _ = bufs[c % 2]
        o_v, osem = obufs[c % 2]
        if c + 1 < _NCH:
            cps[c + 1] = start(c + 1)
        if c - 2 in st_cps:
            st_cps.pop(c - 2).wait()
        cp1, cp2 = cps.pop(c)
        cp1.wait()
        cp2.wait()
        c0 = c * _CH

        def token(t, carry2, r1_v=r1_v, r2_v=r2_v, o_v=o_v, c0=c0):
            g1b = g1_v[pl.ds((c0 + t) * 16, 16)]
            g2b = g2_v[pl.ds((c0 + t) * 16, 16)]
            for h in range(OUT // 16):
                sl = pl.ds(h * 16, 16)
                o_v[t, sl] = g1b * r1_v[t, sl] + g2b * r2_v[t, sl]
            return carry2

        lax.fori_loop(0, _CH, token, 0)
        st_cps[c] = pltpu.async_copy(o_v, out_hbm.at[pl.ds(base + c0, _CH)],
                                     osem)
    for c in sorted(st_cps):
        st_cps[c].wait()


def _combine(eo, gRep, fT):
    gT = gRep.reshape(K, N * 16)
    eo2d = eo.reshape(E * N, OUT)
    mesh = plsc.VectorSubcoreMesh(core_axis_name="c", subcore_axis_name="s")
    k = functools.partial(
        pl.kernel, mesh=mesh,
        out_type=jax.ShapeDtypeStruct((N, OUT), jnp.float32),
        scratch_types=[
            pltpu.VMEM((_BW,), jnp.int32),
            pltpu.VMEM((_BW,), jnp.int32),
            pltpu.VMEM((_BW * 16,), jnp.float32),
            pltpu.VMEM((_BW * 16,), jnp.float32),
            pltpu.VMEM((_CH, OUT), jnp.float32),
            pltpu.VMEM((_CH, OUT), jnp.float32),
            pltpu.VMEM((_CH, OUT), jnp.float32),
            pltpu.VMEM((_CH, OUT), jnp.float32),
            pltpu.VMEM((_CH, OUT), jnp.float32),
            pltpu.VMEM((_CH, OUT), jnp.float32),
            pltpu.SemaphoreType.DMA,
            pltpu.SemaphoreType.DMA,
            pltpu.SemaphoreType.DMA,
            pltpu.SemaphoreType.DMA,
            pltpu.SemaphoreType.DMA,
            pltpu.SemaphoreType.DMA,
        ])(_combine_sc_body)
    return k(eo2d, gT, fT)


def kernel(x, edge_attr, w_gating, W_ep, b_ep, W1, b1, W2, b2, W3, b3):
    xb, gates, idx, loss = _gating(x, edge_attr, w_gating, W_ep, b_ep)
    eo = _experts(xb, W1, b1, W2, b2, W3, b3)
    final = _combine(eo, gates, idx)
    return final, eo, loss[0, 0]
